# Initial kernel scaffold; baseline (speedup 1.0000x reference)
import jax
import jax.numpy as jnp
from jax.experimental import pallas as pl


def kernel(hidden_states, top_k_index, top_k_weights, W_up, W_down):
    raise NotImplementedError("write your pallas kernel here")


# speedup vs baseline: 2.8497x; 2.8497x over previous
import jax
import jax.numpy as jnp
from jax.experimental import pallas as pl
from jax.experimental.pallas import tpu as pltpu

_B = 64
_TB = 64


def _gemm_body(be_ref, bs_ref, bn_ref, rows_ref,
               wts_ref, hs_ref, wup_ref, wdn_ref,
               ys_ref,
               x_ref):
    b = pl.program_id(0)
    n = bn_ref[b]
    start = pl.multiple_of(bs_ref[b], 8)

    @pl.when(n > 0)
    def _():
        for i in range(_B):
            x_ref[i, :] = hs_ref[rows_ref[start + i], :]
        xb = x_ref[...].astype(jnp.bfloat16)
        h = jnp.dot(xb, wup_ref[0].astype(jnp.bfloat16),
                    preferred_element_type=jnp.float32)
        h = jnp.maximum(h, 0.0).astype(jnp.bfloat16)
        y = jnp.dot(h, wdn_ref[0].astype(jnp.bfloat16),
                    preferred_element_type=jnp.float32)
        y = y * wts_ref[pl.ds(start, _B), :]
        ys_ref[pl.ds(start, _B), :] = y


def _combine_body(loc_ref,
                  ys_ref,
                  out_ref):
    g = pl.program_id(0)
    for i in range(_TB):
        t = g * _TB + i
        out_ref[i, :] = ys_ref[loc_ref[2 * t], :] + ys_ref[loc_ref[2 * t + 1], :]


def kernel(hidden_states, top_k_index, top_k_weights, W_up, W_down):
    T, D = hidden_states.shape
    _, K = top_k_index.shape
    E = W_up.shape[0]
    P = T * K
    P8 = P + 8 * E
    MAXB = P // _B + E

    flat_e = top_k_index.reshape(-1)
    sort_idx = jnp.argsort(flat_e).astype(jnp.int32)
    e_sorted = jnp.take(flat_e, sort_idx)
    rows_sorted = (sort_idx // K).astype(jnp.int32)
    wts_sorted = jnp.take(top_k_weights.reshape(-1), sort_idx)
    counts = jnp.zeros((E,), jnp.int32).at[flat_e].add(1)
    starts = jnp.concatenate(
        [jnp.zeros((1,), jnp.int32), jnp.cumsum(counts)[:-1].astype(jnp.int32)])
    counts8 = ((counts + 7) // 8) * 8
    starts8 = jnp.concatenate(
        [jnp.zeros((1,), jnp.int32), jnp.cumsum(counts8)[:-1].astype(jnp.int32)])
    pos_sorted = (jnp.take(starts8, e_sorted) + jnp.arange(P, dtype=jnp.int32)
                  - jnp.take(starts, e_sorted))
    rows_al = jnp.zeros((P8 + _B,), jnp.int32).at[pos_sorted].set(rows_sorted)
    wts_al = jnp.zeros((P8 + _B,), jnp.float32).at[pos_sorted].set(wts_sorted)
    wts_al = wts_al.reshape(P8 + _B, 1)
    nblk = (counts + _B - 1) // _B
    blk_off = jnp.concatenate(
        [jnp.zeros((1,), jnp.int32), jnp.cumsum(nblk)[:-1].astype(jnp.int32)])
    bids = jnp.arange(MAXB, dtype=jnp.int32)
    block_expert = (jnp.searchsorted(blk_off, bids, side='right') - 1).astype(jnp.int32)
    j = bids - jnp.take(blk_off, block_expert)
    block_nrows = jnp.clip(jnp.take(counts, block_expert) - j * _B, 0, _B)
    block_start = jnp.take(starts8, block_expert) + j * _B
    block_start = jnp.where(block_nrows > 0, block_start, 0).astype(jnp.int32)
    inv = jnp.zeros((P,), jnp.int32).at[sort_idx].set(
        jnp.arange(P, dtype=jnp.int32))
    loc = jnp.take(pos_sorted, inv)

    gemm = pl.pallas_call(
        _gemm_body,
        grid_spec=pltpu.PrefetchScalarGridSpec(
            num_scalar_prefetch=4,
            grid=(MAXB,),
            in_specs=[
                pl.BlockSpec((P8 + _B, 1), lambda b, *_: (0, 0)),
                pl.BlockSpec((T, D), lambda b, *_: (0, 0)),
                pl.BlockSpec((1, D, D), lambda b, be, bs, bn, rows: (be[b], 0, 0)),
                pl.BlockSpec((1, D, D), lambda b, be, bs, bn, rows: (be[b], 0, 0)),
            ],
            out_specs=pl.BlockSpec((P8 + _B, D), lambda b, *_: (0, 0)),
            scratch_shapes=[pltpu.VMEM((_B, D), jnp.float32)],
        ),
        out_shape=jax.ShapeDtypeStruct((P8 + _B, D), jnp.float32),
        compiler_params=pltpu.CompilerParams(
            dimension_semantics=("arbitrary",)),
    )
    ys = gemm(block_expert, block_start, block_nrows, rows_al,
              wts_al, hidden_states, W_up, W_down)

    combine = pl.pallas_call(
        _combine_body,
        grid_spec=pltpu.PrefetchScalarGridSpec(
            num_scalar_prefetch=1,
            grid=(T // _TB,),
            in_specs=[pl.BlockSpec((P8 + _B, D), lambda g, loc: (0, 0))],
            out_specs=pl.BlockSpec((_TB, D), lambda g, loc: (g, 0)),
        ),
        out_shape=jax.ShapeDtypeStruct((T, D), jnp.float32),
        compiler_params=pltpu.CompilerParams(
            dimension_semantics=("arbitrary",)),
    )
    out = combine(loc, ys)
    return out.astype(hidden_states.dtype)


# speedup vs baseline: 3.2533x; 1.1416x over previous
import jax
import jax.numpy as jnp
from jax.experimental import pallas as pl
from jax.experimental.pallas import tpu as pltpu

_B = 64
_TB = 64


def _gemm_body(st_ref, cnt_ref, rows_ref,
               wts_ref, hs_ref, wup_ref, wdn_ref,
               ys_ref,
               x_ref):
    e = pl.program_id(0)
    cnt = cnt_ref[e]
    start = st_ref[e]
    nblk = (cnt + _B - 1) // _B

    def blk(j, carry):
        base = pl.multiple_of(start + j * _B, 8)
        for i in range(_B):
            x_ref[i, :] = hs_ref[rows_ref[base + i], :]
        xb = x_ref[...].astype(jnp.bfloat16)
        h = jnp.dot(xb, wup_ref[0].astype(jnp.bfloat16),
                    preferred_element_type=jnp.float32)
        h = jnp.maximum(h, 0.0).astype(jnp.bfloat16)
        y = jnp.dot(h, wdn_ref[0].astype(jnp.bfloat16),
                    preferred_element_type=jnp.float32)
        y = y * wts_ref[pl.ds(base, _B), :]
        ys_ref[pl.ds(base, _B), :] = y
        return carry

    jax.lax.fori_loop(0, nblk, blk, 0)


def _combine_body(loc_ref,
                  ys_ref,
                  out_ref):
    g = pl.program_id(0)
    for i in range(_TB):
        t = g * _TB + i
        out_ref[i, :] = ys_ref[loc_ref[2 * t], :] + ys_ref[loc_ref[2 * t + 1], :]


def kernel(hidden_states, top_k_index, top_k_weights, W_up, W_down):
    T, D = hidden_states.shape
    _, K = top_k_index.shape
    E = W_up.shape[0]
    P = T * K
    P8 = P + 8 * E
    MAXB = P // _B + E

    flat_e = top_k_index.reshape(-1)
    sort_idx = jnp.argsort(flat_e).astype(jnp.int32)
    e_sorted = jnp.take(flat_e, sort_idx)
    rows_sorted = (sort_idx // K).astype(jnp.int32)
    wts_sorted = jnp.take(top_k_weights.reshape(-1), sort_idx)
    counts = jnp.zeros((E,), jnp.int32).at[flat_e].add(1)
    starts = jnp.concatenate(
        [jnp.zeros((1,), jnp.int32), jnp.cumsum(counts)[:-1].astype(jnp.int32)])
    counts8 = ((counts + 7) // 8) * 8
    starts8 = jnp.concatenate(
        [jnp.zeros((1,), jnp.int32), jnp.cumsum(counts8)[:-1].astype(jnp.int32)])
    pos_sorted = (jnp.take(starts8, e_sorted) + jnp.arange(P, dtype=jnp.int32)
                  - jnp.take(starts, e_sorted))
    rows_al = jnp.zeros((P8 + _B,), jnp.int32).at[pos_sorted].set(rows_sorted)
    wts_al = jnp.zeros((P8 + _B,), jnp.float32).at[pos_sorted].set(wts_sorted)
    wts_al = wts_al.reshape(P8 + _B, 1)
    inv = jnp.zeros((P,), jnp.int32).at[sort_idx].set(
        jnp.arange(P, dtype=jnp.int32))
    loc = jnp.take(pos_sorted, inv)

    gemm = pl.pallas_call(
        _gemm_body,
        grid_spec=pltpu.PrefetchScalarGridSpec(
            num_scalar_prefetch=3,
            grid=(E,),
            in_specs=[
                pl.BlockSpec((P8 + _B, 1), lambda e, *_: (0, 0)),
                pl.BlockSpec((T, D), lambda e, *_: (0, 0)),
                pl.BlockSpec((1, D, D), lambda e, *_: (e, 0, 0)),
                pl.BlockSpec((1, D, D), lambda e, *_: (e, 0, 0)),
            ],
            out_specs=pl.BlockSpec((P8 + _B, D), lambda e, *_: (0, 0)),
            scratch_shapes=[pltpu.VMEM((_B, D), jnp.float32)],
        ),
        out_shape=jax.ShapeDtypeStruct((P8 + _B, D), jnp.float32),
        compiler_params=pltpu.CompilerParams(
            dimension_semantics=("arbitrary",)),
    )
    ys = gemm(starts8, counts, rows_al, wts_al, hidden_states, W_up, W_down)

    combine = pl.pallas_call(
        _combine_body,
        grid_spec=pltpu.PrefetchScalarGridSpec(
            num_scalar_prefetch=1,
            grid=(T // _TB,),
            in_specs=[pl.BlockSpec((P8 + _B, D), lambda g, loc: (0, 0))],
            out_specs=pl.BlockSpec((_TB, D), lambda g, loc: (g, 0)),
        ),
        out_shape=jax.ShapeDtypeStruct((T, D), jnp.float32),
        compiler_params=pltpu.CompilerParams(
            dimension_semantics=("arbitrary",)),
    )
    out = combine(loc, ys)
    return out.astype(hidden_states.dtype)


# speedup vs baseline: 4.1338x; 1.2706x over previous
import functools

import jax
import jax.numpy as jnp
from jax.experimental import pallas as pl
from jax.experimental.pallas import tpu as pltpu

_B = 64
_TB = 64


def _meta_body(E, K, P, tki_ref, tkw_ref,
               counts_ref, starts8_ref, rows_ref, wts_ref, loc_ref,
               cur_ref):
    def zero(e, c):
        counts_ref[e] = 0
        return c

    jax.lax.fori_loop(0, E, zero, jnp.int32(0))

    def count(p, c):
        e = tki_ref[p]
        counts_ref[e] = counts_ref[e] + 1
        return c

    jax.lax.fori_loop(0, P, count, jnp.int32(0))

    def offsets(e, acc):
        starts8_ref[e] = acc
        cur_ref[e] = 0
        cnt = counts_ref[e]
        return acc + ((cnt + 7) // 8) * 8

    jax.lax.fori_loop(0, E, offsets, jnp.int32(0))

    def place(p, c):
        e = tki_ref[p]
        pos = starts8_ref[e] + cur_ref[e]
        cur_ref[e] = cur_ref[e] + 1
        rows_ref[pos] = p // K
        wts_ref[pos] = tkw_ref[p]
        loc_ref[p] = pos
        return c

    jax.lax.fori_loop(0, P, place, jnp.int32(0))


def _gemm_body(T, st_ref, cnt_ref, rows_ref,
               wts_ref, hs_ref, wup_ref, wdn_ref,
               ys_ref,
               x_ref):
    e = pl.program_id(0)
    cnt = cnt_ref[e]
    start = st_ref[e]
    nblk = (cnt + _B - 1) // _B

    def blk(j, carry):
        base = pl.multiple_of(start + j * _B, 8)
        for i in range(_B):
            r = jnp.clip(rows_ref[base + i], 0, T - 1)
            x_ref[i, :] = hs_ref[r, :]
        xb = x_ref[...].astype(jnp.bfloat16)
        h = jnp.dot(xb, wup_ref[0].astype(jnp.bfloat16),
                    preferred_element_type=jnp.float32)
        h = jnp.maximum(h, 0.0).astype(jnp.bfloat16)
        y = jnp.dot(h, wdn_ref[0].astype(jnp.bfloat16),
                    preferred_element_type=jnp.float32)
        y = y * wts_ref[pl.ds(base, _B), :]
        ys_ref[pl.ds(base, _B), :] = y
        return carry

    jax.lax.fori_loop(0, nblk, blk, 0)


def _combine_body(loc_ref,
                  ys_ref,
                  out_ref):
    g = pl.program_id(0)
    for i in range(_TB):
        t = g * _TB + i
        out_ref[i, :] = ys_ref[loc_ref[2 * t], :] + ys_ref[loc_ref[2 * t + 1], :]


def kernel(hidden_states, top_k_index, top_k_weights, W_up, W_down):
    T, D = hidden_states.shape
    _, K = top_k_index.shape
    E = W_up.shape[0]
    P = T * K
    P8 = P + 8 * E

    meta = pl.pallas_call(
        functools.partial(_meta_body, E, K, P),
        in_specs=[
            pl.BlockSpec(memory_space=pltpu.SMEM),
            pl.BlockSpec(memory_space=pltpu.SMEM),
        ],
        out_specs=[
            pl.BlockSpec(memory_space=pltpu.SMEM),
            pl.BlockSpec(memory_space=pltpu.SMEM),
            pl.BlockSpec(memory_space=pltpu.SMEM),
            pl.BlockSpec(memory_space=pltpu.SMEM),
            pl.BlockSpec(memory_space=pltpu.SMEM),
        ],
        out_shape=[
            jax.ShapeDtypeStruct((E,), jnp.int32),
            jax.ShapeDtypeStruct((E,), jnp.int32),
            jax.ShapeDtypeStruct((P8 + _B,), jnp.int32),
            jax.ShapeDtypeStruct((P8 + _B,), jnp.float32),
            jax.ShapeDtypeStruct((P,), jnp.int32),
        ],
        scratch_shapes=[pltpu.SMEM((E,), jnp.int32)],
    )
    counts, starts8, rows_al, wts_al, loc = meta(
        top_k_index.reshape(P), top_k_weights.reshape(P))
    wts_al = wts_al.reshape(P8 + _B, 1)

    gemm = pl.pallas_call(
        functools.partial(_gemm_body, T),
        grid_spec=pltpu.PrefetchScalarGridSpec(
            num_scalar_prefetch=3,
            grid=(E,),
            in_specs=[
                pl.BlockSpec((P8 + _B, 1), lambda e, *_: (0, 0)),
                pl.BlockSpec((T, D), lambda e, *_: (0, 0)),
                pl.BlockSpec((1, D, D), lambda e, *_: (e, 0, 0)),
                pl.BlockSpec((1, D, D), lambda e, *_: (e, 0, 0)),
            ],
            out_specs=pl.BlockSpec((P8 + _B, D), lambda e, *_: (0, 0)),
            scratch_shapes=[pltpu.VMEM((_B, D), jnp.float32)],
        ),
        out_shape=jax.ShapeDtypeStruct((P8 + _B, D), jnp.float32),
        compiler_params=pltpu.CompilerParams(
            dimension_semantics=("arbitrary",)),
    )
    ys = gemm(starts8, counts, rows_al, wts_al, hidden_states, W_up, W_down)

    combine = pl.pallas_call(
        _combine_body,
        grid_spec=pltpu.PrefetchScalarGridSpec(
            num_scalar_prefetch=1,
            grid=(T // _TB,),
            in_specs=[pl.BlockSpec((P8 + _B, D), lambda g, loc: (0, 0))],
            out_specs=pl.BlockSpec((_TB, D), lambda g, loc: (g, 0)),
        ),
        out_shape=jax.ShapeDtypeStruct((T, D), jnp.float32),
        compiler_params=pltpu.CompilerParams(
            dimension_semantics=("arbitrary",)),
    )
    out = combine(loc, ys)
    return out.astype(hidden_states.dtype)


# speedup vs baseline: 4.5214x; 1.0938x over previous
import functools

import jax
import jax.numpy as jnp
from jax.experimental import pallas as pl
from jax.experimental.pallas import tpu as pltpu

_B = 64
_TB = 64


def _metav_body(E, C, tki_ref, loc_ref, counts_ref, starts8_ref):
    P = tki_ref.shape[0]
    NC = P // C
    lane_e = jax.lax.broadcasted_iota(jnp.int32, (C, E), 1)
    row_i = jax.lax.broadcasted_iota(jnp.int32, (C, C), 0)
    col_i = jax.lax.broadcasted_iota(jnp.int32, (C, C), 1)
    ltri = (row_i > col_i).astype(jnp.bfloat16)

    def count_chunk(c, acc):
        ech = tki_ref[pl.ds(c * C, C), :]
        oh = (ech == lane_e).astype(jnp.float32)
        return acc + jnp.sum(oh, axis=0, keepdims=True)

    counts_f = jax.lax.fori_loop(
        0, NC, count_chunk, jnp.zeros((1, E), jnp.float32))
    counts = counts_f.astype(jnp.int32)
    counts8 = ((counts + 7) >> 3) << 3
    lane_idx = jax.lax.broadcasted_iota(jnp.int32, (1, E), 1)
    acc = counts8
    sh = 1
    while sh < E:
        r = pltpu.roll(acc, sh, 1)
        acc = acc + jnp.where(lane_idx >= sh, r, 0)
        sh *= 2
    starts8 = acc - counts8
    counts_ref[...] = counts
    starts8_ref[...] = starts8
    base_f = starts8.astype(jnp.float32)

    def place_chunk(c, cum):
        ech = tki_ref[pl.ds(c * C, C), :]
        ohf = (ech == lane_e).astype(jnp.float32)
        rank = jnp.dot(ltri, ohf.astype(jnp.bfloat16),
                       preferred_element_type=jnp.float32)
        pos = jnp.sum(ohf * (base_f + cum + rank), axis=1, keepdims=True)
        loc_ref[pl.ds(c * C, C), :] = pos.astype(jnp.int32)
        return cum + jnp.sum(ohf, axis=0, keepdims=True)

    jax.lax.fori_loop(0, NC, place_chunk, jnp.zeros((1, E), jnp.float32))


def _metas_body(K, loc_ref, tkw_ref, rows_ref, wts_ref):
    P = loc_ref.shape[0]

    def place(p, c):
        pos = loc_ref[p]
        rows_ref[pos] = p // K
        wts_ref[pos] = tkw_ref[p]
        return c

    jax.lax.fori_loop(0, P, place, jnp.int32(0))


def _gemm_body(T, st_ref, cnt_ref, rows_ref,
               wts_ref, hs_ref, wup_ref, wdn_ref,
               ys_ref,
               x_ref):
    e = pl.program_id(0)
    cnt = cnt_ref[e]
    start = st_ref[e]
    nblk = (cnt + _B - 1) // _B

    def blk(j, carry):
        base = pl.multiple_of(start + j * _B, 8)
        for i in range(_B):
            r = jnp.clip(rows_ref[base + i], 0, T - 1)
            x_ref[i, :] = hs_ref[r, :]
        xb = x_ref[...].astype(jnp.bfloat16)
        h = jnp.dot(xb, wup_ref[0].astype(jnp.bfloat16),
                    preferred_element_type=jnp.float32)
        h = jnp.maximum(h, 0.0).astype(jnp.bfloat16)
        y = jnp.dot(h, wdn_ref[0].astype(jnp.bfloat16),
                    preferred_element_type=jnp.float32)
        y = y * wts_ref[pl.ds(base, _B), :]
        ys_ref[pl.ds(base, _B), :] = y
        return carry

    jax.lax.fori_loop(0, nblk, blk, 0)


def _combine_body(loc_ref,
                  ys_ref,
                  out_ref):
    g = pl.program_id(0)
    for i in range(_TB):
        t = g * _TB + i
        out_ref[i, :] = ys_ref[loc_ref[2 * t], :] + ys_ref[loc_ref[2 * t + 1], :]


def kernel(hidden_states, top_k_index, top_k_weights, W_up, W_down):
    T, D = hidden_states.shape
    _, K = top_k_index.shape
    E = W_up.shape[0]
    P = T * K
    P8 = P + 8 * E

    metav = pl.pallas_call(
        functools.partial(_metav_body, E, 256),
        out_shape=[
            jax.ShapeDtypeStruct((P, 1), jnp.int32),
            jax.ShapeDtypeStruct((1, E), jnp.int32),
            jax.ShapeDtypeStruct((1, E), jnp.int32),
        ],
    )
    loc_col, counts_row, starts8_row = metav(top_k_index.reshape(P, 1))
    loc = loc_col.reshape(P)
    counts = counts_row.reshape(E)
    starts8 = starts8_row.reshape(E)

    metas = pl.pallas_call(
        functools.partial(_metas_body, K),
        in_specs=[
            pl.BlockSpec(memory_space=pltpu.SMEM),
            pl.BlockSpec(memory_space=pltpu.SMEM),
        ],
        out_specs=[
            pl.BlockSpec(memory_space=pltpu.SMEM),
            pl.BlockSpec(memory_space=pltpu.SMEM),
        ],
        out_shape=[
            jax.ShapeDtypeStruct((P8 + _B,), jnp.int32),
            jax.ShapeDtypeStruct((P8 + _B,), jnp.float32),
        ],
    )
    rows_al, wts_al = metas(loc, top_k_weights.reshape(P))
    wts_al = wts_al.reshape(P8 + _B, 1)

    gemm = pl.pallas_call(
        functools.partial(_gemm_body, T),
        grid_spec=pltpu.PrefetchScalarGridSpec(
            num_scalar_prefetch=3,
            grid=(E,),
            in_specs=[
                pl.BlockSpec((P8 + _B, 1), lambda e, *_: (0, 0)),
                pl.BlockSpec((T, D), lambda e, *_: (0, 0)),
                pl.BlockSpec((1, D, D), lambda e, *_: (e, 0, 0)),
                pl.BlockSpec((1, D, D), lambda e, *_: (e, 0, 0)),
            ],
            out_specs=pl.BlockSpec((P8 + _B, D), lambda e, *_: (0, 0)),
            scratch_shapes=[pltpu.VMEM((_B, D), jnp.float32)],
        ),
        out_shape=jax.ShapeDtypeStruct((P8 + _B, D), jnp.float32),
        compiler_params=pltpu.CompilerParams(
            dimension_semantics=("arbitrary",)),
    )
    ys = gemm(starts8, counts, rows_al, wts_al, hidden_states, W_up, W_down)

    combine = pl.pallas_call(
        _combine_body,
        grid_spec=pltpu.PrefetchScalarGridSpec(
            num_scalar_prefetch=1,
            grid=(T // _TB,),
            in_specs=[pl.BlockSpec((P8 + _B, D), lambda g, loc: (0, 0))],
            out_specs=pl.BlockSpec((_TB, D), lambda g, loc: (g, 0)),
        ),
        out_shape=jax.ShapeDtypeStruct((T, D), jnp.float32),
        compiler_params=pltpu.CompilerParams(
            dimension_semantics=("arbitrary",)),
    )
    out = combine(loc, ys)
    return out.astype(hidden_states.dtype)


# speedup vs baseline: 4.5908x; 1.0153x over previous
import functools

import jax
import jax.numpy as jnp
from jax.experimental import pallas as pl
from jax.experimental.pallas import tpu as pltpu

_B = 64
_TB = 64


def _metav_body(E, C, tki_ref, loc_ref, counts_ref, starts8_ref):
    NC = tki_ref.shape[0]
    sub_e = jax.lax.broadcasted_iota(jnp.int32, (E, C), 0)
    row_i = jax.lax.broadcasted_iota(jnp.int32, (C, C), 0)
    col_i = jax.lax.broadcasted_iota(jnp.int32, (C, C), 1)
    utri = (row_i < col_i).astype(jnp.bfloat16)

    def count_chunk(c, acc):
        ech = tki_ref[pl.ds(c, 1), :]
        oh = (ech == sub_e).astype(jnp.float32)
        return acc + jnp.sum(oh, axis=1, keepdims=True)

    counts_f = jax.lax.fori_loop(
        0, NC, count_chunk, jnp.zeros((E, 1), jnp.float32))
    counts = counts_f.astype(jnp.int32)
    counts8 = ((counts + 7) >> 3) << 3
    sub_idx = jax.lax.broadcasted_iota(jnp.int32, (E, 1), 0)
    acc = counts8
    sh = 1
    while sh < E:
        r = pltpu.roll(acc, sh, 0)
        acc = acc + jnp.where(sub_idx >= sh, r, 0)
        sh *= 2
    starts8 = acc - counts8
    counts_ref[...] = counts
    starts8_ref[...] = starts8
    base_f = starts8.astype(jnp.float32)

    def place_chunk(c, cum):
        ech = tki_ref[pl.ds(c, 1), :]
        ohf = (ech == sub_e).astype(jnp.float32)
        rank = jnp.dot(ohf.astype(jnp.bfloat16), utri,
                       preferred_element_type=jnp.float32)
        pos = jnp.sum(ohf * (base_f + cum + rank), axis=0, keepdims=True)
        loc_ref[pl.ds(c, 1), :] = pos.astype(jnp.int32)
        return cum + jnp.sum(ohf, axis=1, keepdims=True)

    jax.lax.fori_loop(0, NC, place_chunk, jnp.zeros((E, 1), jnp.float32))


def _metas_body(K, loc_ref, tkw_ref, rows_ref, wts_ref):
    P = loc_ref.shape[0]

    def place(p, c):
        pos = loc_ref[p]
        rows_ref[pos] = p // K
        wts_ref[pos] = tkw_ref[p]
        return c

    jax.lax.fori_loop(0, P, place, jnp.int32(0))


def _gemm_body(T, st_ref, cnt_ref, rows_ref,
               wts_ref, hs_ref, wup_ref, wdn_ref,
               ys_ref,
               x_ref):
    e = pl.program_id(0)
    cnt = cnt_ref[e]
    start = st_ref[e]
    nblk = (cnt + _B - 1) // _B

    def blk(j, carry):
        base = pl.multiple_of(start + j * _B, 8)
        for i in range(_B):
            r = jnp.clip(rows_ref[base + i], 0, T - 1)
            x_ref[i, :] = hs_ref[r, :]
        xb = x_ref[...].astype(jnp.bfloat16)
        h = jnp.dot(xb, wup_ref[0].astype(jnp.bfloat16),
                    preferred_element_type=jnp.float32)
        h = jnp.maximum(h, 0.0).astype(jnp.bfloat16)
        y = jnp.dot(h, wdn_ref[0].astype(jnp.bfloat16),
                    preferred_element_type=jnp.float32)
        y = y * wts_ref[pl.ds(base, _B), :]
        ys_ref[pl.ds(base, _B), :] = y
        return carry

    jax.lax.fori_loop(0, nblk, blk, 0)


def _combine_body(loc_ref,
                  ys_ref,
                  out_ref):
    g = pl.program_id(0)
    for i in range(_TB):
        t = g * _TB + i
        out_ref[i, :] = ys_ref[loc_ref[2 * t], :] + ys_ref[loc_ref[2 * t + 1], :]


def kernel(hidden_states, top_k_index, top_k_weights, W_up, W_down):
    T, D = hidden_states.shape
    _, K = top_k_index.shape
    E = W_up.shape[0]
    P = T * K
    P8 = P + 8 * E

    C = 256
    NC = P // C
    metav = pl.pallas_call(
        functools.partial(_metav_body, E, C),
        out_shape=[
            jax.ShapeDtypeStruct((NC, C), jnp.int32),
            jax.ShapeDtypeStruct((E, 1), jnp.int32),
            jax.ShapeDtypeStruct((E, 1), jnp.int32),
        ],
    )
    loc_col, counts_row, starts8_row = metav(top_k_index.reshape(NC, C))
    loc = loc_col.reshape(P)
    counts = counts_row.reshape(E)
    starts8 = starts8_row.reshape(E)

    metas = pl.pallas_call(
        functools.partial(_metas_body, K),
        in_specs=[
            pl.BlockSpec(memory_space=pltpu.SMEM),
            pl.BlockSpec(memory_space=pltpu.SMEM),
        ],
        out_specs=[
            pl.BlockSpec(memory_space=pltpu.SMEM),
            pl.BlockSpec(memory_space=pltpu.SMEM),
        ],
        out_shape=[
            jax.ShapeDtypeStruct((P8 + _B,), jnp.int32),
            jax.ShapeDtypeStruct((P8 + _B,), jnp.float32),
        ],
    )
    rows_al, wts_al = metas(loc, top_k_weights.reshape(P))
    wts_al = wts_al.reshape(P8 + _B, 1)

    gemm = pl.pallas_call(
        functools.partial(_gemm_body, T),
        grid_spec=pltpu.PrefetchScalarGridSpec(
            num_scalar_prefetch=3,
            grid=(E,),
            in_specs=[
                pl.BlockSpec((P8 + _B, 1), lambda e, *_: (0, 0)),
                pl.BlockSpec((T, D), lambda e, *_: (0, 0)),
                pl.BlockSpec((1, D, D), lambda e, *_: (e, 0, 0)),
                pl.BlockSpec((1, D, D), lambda e, *_: (e, 0, 0)),
            ],
            out_specs=pl.BlockSpec((P8 + _B, D), lambda e, *_: (0, 0)),
            scratch_shapes=[pltpu.VMEM((_B, D), jnp.float32)],
        ),
        out_shape=jax.ShapeDtypeStruct((P8 + _B, D), jnp.float32),
        compiler_params=pltpu.CompilerParams(
            dimension_semantics=("arbitrary",)),
    )
    ys = gemm(starts8, counts, rows_al, wts_al, hidden_states, W_up, W_down)

    combine = pl.pallas_call(
        _combine_body,
        grid_spec=pltpu.PrefetchScalarGridSpec(
            num_scalar_prefetch=1,
            grid=(T // _TB,),
            in_specs=[pl.BlockSpec((P8 + _B, D), lambda g, loc: (0, 0))],
            out_specs=pl.BlockSpec((_TB, D), lambda g, loc: (g, 0)),
        ),
        out_shape=jax.ShapeDtypeStruct((T, D), jnp.float32),
        compiler_params=pltpu.CompilerParams(
            dimension_semantics=("arbitrary",)),
    )
    out = combine(loc, ys)
    return out.astype(hidden_states.dtype)


# speedup vs baseline: 5.1336x; 1.1182x over previous
import functools

import jax
import jax.numpy as jnp
from jax.experimental import pallas as pl
from jax.experimental.pallas import tpu as pltpu

_B = 64
_TB = 64


def _metav_body(E, C, tki_ref, loc_ref, counts_ref, starts8_ref):
    NC = tki_ref.shape[0]
    sub_e = jax.lax.broadcasted_iota(jnp.int32, (E, C), 0)
    row_i = jax.lax.broadcasted_iota(jnp.int32, (C, C), 0)
    col_i = jax.lax.broadcasted_iota(jnp.int32, (C, C), 1)
    utri = (row_i < col_i).astype(jnp.bfloat16)

    def count_chunk(c, acc):
        ech = tki_ref[pl.ds(c, 1), :]
        oh = (ech == sub_e).astype(jnp.float32)
        return acc + jnp.sum(oh, axis=1, keepdims=True)

    counts_f = jax.lax.fori_loop(
        0, NC, count_chunk, jnp.zeros((E, 1), jnp.float32))
    counts = counts_f.astype(jnp.int32)
    counts8 = ((counts + 7) >> 3) << 3
    sub_idx = jax.lax.broadcasted_iota(jnp.int32, (E, 1), 0)
    acc = counts8
    sh = 1
    while sh < E:
        r = pltpu.roll(acc, sh, 0)
        acc = acc + jnp.where(sub_idx >= sh, r, 0)
        sh *= 2
    starts8 = acc - counts8
    counts_ref[...] = counts
    starts8_ref[...] = starts8
    base_f = starts8.astype(jnp.float32)

    def place_chunk(c, cum):
        ech = tki_ref[pl.ds(c, 1), :]
        ohf = (ech == sub_e).astype(jnp.float32)
        rank = jnp.dot(ohf.astype(jnp.bfloat16), utri,
                       preferred_element_type=jnp.float32)
        pos = jnp.sum(ohf * (base_f + cum + rank), axis=0, keepdims=True)
        loc_ref[pl.ds(c, 1), :] = pos.astype(jnp.int32)
        return cum + jnp.sum(ohf, axis=1, keepdims=True)

    jax.lax.fori_loop(0, NC, place_chunk, jnp.zeros((E, 1), jnp.float32))


def _gemm_body(T, K,
               st_ref, cnt_ref, loc_ref, tkw_ref,
               hs_ref, wup_ref, wdn_ref,
               ys_ref, out_ref,
               x_ref, rows_ref):
    e = pl.program_id(0)
    num_e = pl.num_programs(0)
    P = loc_ref.shape[0]

    @pl.when(e == 0)
    def _():
        def place(p, c):
            rows_ref[loc_ref[p]] = p // K
            return c

        jax.lax.fori_loop(0, P, place, jnp.int32(0))

    cnt = cnt_ref[e]
    start = st_ref[e]
    nblk = (cnt + _B - 1) // _B

    def blk(j, carry):
        base = pl.multiple_of(start + j * _B, 8)
        for i in range(_B):
            r = jnp.clip(rows_ref[base + i], 0, T - 1)
            x_ref[i, :] = hs_ref[r, :]
        xb = x_ref[...].astype(jnp.bfloat16)
        h = jnp.dot(xb, wup_ref[0].astype(jnp.bfloat16),
                    preferred_element_type=jnp.float32)
        h = jnp.maximum(h, 0.0).astype(jnp.bfloat16)
        y = jnp.dot(h, wdn_ref[0].astype(jnp.bfloat16),
                    preferred_element_type=jnp.float32)
        ys_ref[pl.ds(base, _B), :] = y
        return carry

    jax.lax.fori_loop(0, nblk, blk, 0)

    @pl.when(e == num_e - 1)
    def _():
        def group(g, carry):
            for i in range(_TB):
                t = g * _TB + i
                acc = ys_ref[loc_ref[K * t], :] * tkw_ref[K * t]
                for k in range(1, K):
                    acc = acc + ys_ref[loc_ref[K * t + k], :] * tkw_ref[K * t + k]
                x_ref[i, :] = acc
            out_ref[pl.ds(pl.multiple_of(g * _TB, 8), _TB), :] = x_ref[...]
            return carry

        jax.lax.fori_loop(0, T // _TB, group, jnp.int32(0))


def kernel(hidden_states, top_k_index, top_k_weights, W_up, W_down):
    T, D = hidden_states.shape
    _, K = top_k_index.shape
    E = W_up.shape[0]
    P = T * K
    P8 = P + 8 * E

    C = 256
    NC = P // C
    metav = pl.pallas_call(
        functools.partial(_metav_body, E, C),
        out_shape=[
            jax.ShapeDtypeStruct((NC, C), jnp.int32),
            jax.ShapeDtypeStruct((E, 1), jnp.int32),
            jax.ShapeDtypeStruct((E, 1), jnp.int32),
        ],
    )
    loc_col, counts_row, starts8_row = metav(top_k_index.reshape(NC, C))
    loc = loc_col.reshape(P)
    counts = counts_row.reshape(E)
    starts8 = starts8_row.reshape(E)

    gemm = pl.pallas_call(
        functools.partial(_gemm_body, T, K),
        grid_spec=pltpu.PrefetchScalarGridSpec(
            num_scalar_prefetch=4,
            grid=(E,),
            in_specs=[
                pl.BlockSpec((T, D), lambda e, *_: (0, 0)),
                pl.BlockSpec((1, D, D), lambda e, *_: (e, 0, 0)),
                pl.BlockSpec((1, D, D), lambda e, *_: (e, 0, 0)),
            ],
            out_specs=[
                pl.BlockSpec((P8 + _B, D), lambda e, *_: (0, 0)),
                pl.BlockSpec((T, D), lambda e, *_: (0, 0)),
            ],
            scratch_shapes=[pltpu.VMEM((_B, D), jnp.float32),
                            pltpu.SMEM((P8 + _B,), jnp.int32)],
        ),
        out_shape=[
            jax.ShapeDtypeStruct((P8 + _B, D), jnp.float32),
            jax.ShapeDtypeStruct((T, D), jnp.float32),
        ],
        compiler_params=pltpu.CompilerParams(
            dimension_semantics=("arbitrary",)),
    )
    _, out = gemm(starts8, counts, loc, top_k_weights.reshape(P),
                  hidden_states, W_up, W_down)
    return out.astype(hidden_states.dtype)
